# baseline (speedup 1.0000x reference)
import functools

import jax
import jax.numpy as jnp
from jax.experimental import pallas as pl
from jax.experimental.pallas import tpu as pltpu

B, N, D_IN, D_E, OUT, TGT = 64, 32, 128, 16, 128, 12
NDEG = 33
P = B * N
MSG = D_IN + D_E
EW = N * D_E
CH = 256
C_LO, C_HI = 8, 24

_F32 = jnp.float32
_BF16 = jnp.bfloat16


def _dot(a, b):
    return jax.lax.dot_general(
        a, b, (((1,), (0,)), ((), ())), preferred_element_type=_F32)


def _gnn_kernel(g_ref, e_ref, h_ref, H0_ref, H1_ref, H2_ref,
                W0_ref, W1_ref, W2_ref, W3_ref,
                nW0_ref, nb0_ref, nW1_ref, nb1_ref,
                nW2_ref, nb2_ref, nW3_ref, nb3_ref,
                out_ref, gbd_ref, acc_ref):
    g = g_ref[...]
    deg = jnp.sum(g, axis=1, keepdims=True)
    deg = jnp.minimum(deg, float(NDEG - 1))
    deg_min = jnp.min(deg)
    deg_max = jnp.max(deg)

    tq = jax.lax.broadcasted_iota(jnp.int32, (N, CH), 1)
    tw = jax.lax.broadcasted_iota(jnp.int32, (N, CH), 0)
    T = (tq % N == tw).astype(_BF16)
    ri = jax.lax.broadcasted_iota(jnp.int32, (CH, CH), 0)
    ci = jax.lax.broadcasted_iota(jnp.int32, (CH, CH), 1)
    blk = (ri // N == ci // N).astype(_BF16)
    for c in range(P // CH):
        rows = _dot(g_ref[c * CH:(c + 1) * CH, :].astype(_BF16), T)
        gbd_ref[c * CH:(c + 1) * CH, :] = rows.astype(_BF16) * blk

    rl = jax.lax.broadcasted_iota(jnp.int32, (N, EW), 1)
    rw = jax.lax.broadcasted_iota(jnp.int32, (N, EW), 0)
    R = (rl // D_E == rw).astype(_BF16)
    sl = jax.lax.broadcasted_iota(jnp.int32, (EW, D_E), 0)
    sj = jax.lax.broadcasted_iota(jnp.int32, (EW, D_E), 1)
    S = (sl % D_E == sj).astype(_BF16)
    g_rep = _dot(g.astype(_BF16), R)
    prod = g_rep.astype(_BF16) * e_ref[...].astype(_BF16)
    m_e = _dot(prod, S)
    m_e_bf = m_e.astype(_BF16)

    dmask = [(deg == float(d)).astype(_BF16) for d in range(NDEG)]

    el = jax.lax.broadcasted_iota(jnp.int32, (D_E, 8 * D_E), 1)
    ei = jax.lax.broadcasted_iota(jnp.int32, (D_E, 8 * D_E), 0)
    Erep = (el % D_E == ei).astype(_BF16)
    qpat = (jax.lax.broadcasted_iota(jnp.int32, (1, 8 * D_E), 1)
            // D_E).astype(_F32)

    m_e8 = _dot(m_e_bf, Erep).astype(_BF16)
    epiece = [m_e8 * (deg - float(8 * q) == qpat).astype(_BF16)
              for q in range(4)]
    e32 = m_e_bf * (deg == float(32)).astype(_BF16)

    def readout(h_l, W_ref):
        mask = (jnp.sum(h_l, axis=1, keepdims=True) != 0).astype(_F32)
        hm = h_l * mask
        hsum = jnp.sum(hm.reshape(B, N, OUT), axis=1)
        return _dot(hsum, W_ref[...])

    h = h_ref[...]
    aux = readout(h, W0_ref)

    for H_ref, W_ref in ((H0_ref, W1_ref), (H1_ref, W2_ref), (H2_ref, W3_ref)):
        h_bf = h.astype(_BF16)
        mh_cs = [_dot(gbd_ref[c * CH:(c + 1) * CH, :],
                      h_bf[c * CH:(c + 1) * CH, :]) for c in range(P // CH)]
        m_h = jnp.concatenate(mh_cs, axis=0)
        m_h_bf = m_h.astype(_BF16)

        def hpieces(lo, hi):
            return [m_h_bf * dmask[d] for d in range(lo, hi)]

        def hbank(ref, lo, hi):
            return jnp.concatenate(
                [ref[d, :D_IN, :] for d in range(lo, hi)],
                axis=0).astype(_BF16)

        def ebank(ref, lo, hi):
            return jnp.concatenate(
                [ref[d, D_IN:, :] for d in range(lo, hi)],
                axis=0).astype(_BF16)

        lhs_c = jnp.concatenate(
            hpieces(C_LO, C_HI) + [epiece[1], epiece[2]], axis=1)
        rhs_c = jnp.concatenate(
            [hbank(H_ref, C_LO, C_HI), ebank(H_ref, C_LO, C_HI)], axis=0)
        acc_ref[...] = _dot(lhs_c, rhs_c)

        @pl.when(deg_min < float(C_LO))
        def _():
            lhs = jnp.concatenate(hpieces(0, C_LO) + [epiece[0]], axis=1)
            rhs = jnp.concatenate(
                [hbank(H_ref, 0, C_LO), ebank(H_ref, 0, C_LO)], axis=0)
            acc_ref[...] += _dot(lhs, rhs)

        @pl.when(deg_max >= float(C_HI))
        def _():
            lhs = jnp.concatenate(hpieces(C_HI, 32) + [epiece[3]], axis=1)
            rhs = jnp.concatenate(
                [hbank(H_ref, C_HI, 32), ebank(H_ref, C_HI, 32)], axis=0)
            acc_ref[...] += _dot(lhs, rhs)

        @pl.when(deg_max == float(32))
        def _():
            lhs = jnp.concatenate(
                [m_h_bf * dmask[32], e32], axis=1)
            acc_ref[...] += _dot(lhs, H_ref[32].astype(_BF16))

        h = jax.nn.sigmoid(acc_ref[...])
        aux = aux + readout(h, W_ref)

    s = jax.nn.softmax(aux, axis=1)
    x = jax.nn.relu(_dot(s, nW0_ref[...]) + nb0_ref[...])
    x = jax.nn.relu(_dot(x, nW1_ref[...]) + nb1_ref[...])
    x = jax.nn.relu(_dot(x, nW2_ref[...]) + nb2_ref[...])
    x = jax.nn.sigmoid(_dot(x, nW3_ref[...]) + nb3_ref[...])
    out_ref[...] = jax.nn.softmax(x, axis=1)


@functools.partial(jax.jit, static_argnames=("interpret",))
def _run(g, h_in, e, H0, H1, H2, W0, W1, W2, W3,
         nW0, nb0, nW1, nb1, nW2, nb2, nW3, nb3, interpret=False):
    g2 = g.reshape(P, N)
    e2 = e.reshape(P, EW)
    h2 = h_in.reshape(P, D_IN)
    return pl.pallas_call(
        _gnn_kernel,
        out_shape=jax.ShapeDtypeStruct((B, TGT), _F32),
        scratch_shapes=[pltpu.VMEM((P, CH), _BF16),
                        pltpu.VMEM((P, OUT), _F32)],
        interpret=interpret,
    )(g2, e2, h2, H0, H1, H2, W0, W1, W2, W3,
      nW0, nb0.reshape(1, -1), nW1, nb1.reshape(1, -1),
      nW2, nb2.reshape(1, -1), nW3, nb3.reshape(1, -1))


def kernel(g, h_in, e, H0, H1, H2, W0, W1, W2, W3,
           nW0, nb0, nW1, nb1, nW2, nb2, nW3, nb3):
    return _run(g, h_in, e, H0, H1, H2, W0, W1, W2, W3,
                nW0, nb0, nW1, nb1, nW2, nb2, nW3, nb3)


# speedup vs baseline: 1.0130x; 1.0130x over previous
import functools

import jax
import jax.numpy as jnp
from jax.experimental import pallas as pl
from jax.experimental.pallas import tpu as pltpu

B, N, D_IN, D_E, OUT, TGT = 64, 32, 128, 16, 128, 12
NDEG = 33
P = B * N
MSG = D_IN + D_E
EW = N * D_E
CH = 256
C_LO, C_HI = 8, 24

_F32 = jnp.float32
_BF16 = jnp.bfloat16


def _dot(a, b):
    return jax.lax.dot_general(
        a, b, (((1,), (0,)), ((), ())), preferred_element_type=_F32)


def _gnn_kernel(g_ref, e_ref, h_ref, H0_ref, H1_ref, H2_ref,
                W0_ref, W1_ref, W2_ref, W3_ref,
                nW0_ref, nb0_ref, nW1_ref, nb1_ref,
                nW2_ref, nb2_ref, nW3_ref, nb3_ref,
                out_ref, gbd_ref, acc_ref):
    g = g_ref[...]
    deg = jnp.sum(g, axis=1, keepdims=True)
    deg = jnp.minimum(deg, float(NDEG - 1))
    deg_min = jnp.min(deg)
    deg_max = jnp.max(deg)
    one = jnp.ones_like(deg)
    zero = jnp.zeros_like(deg)
    any47 = jnp.max(jnp.where((deg >= 4.0) & (deg < 8.0), one, zero))
    any2427 = jnp.max(jnp.where((deg >= 24.0) & (deg < 28.0), one, zero))
    any2831 = jnp.max(jnp.where((deg >= 28.0) & (deg < 32.0), one, zero))

    tq = jax.lax.broadcasted_iota(jnp.int32, (N, CH), 1)
    tw = jax.lax.broadcasted_iota(jnp.int32, (N, CH), 0)
    T = (tq % N == tw).astype(_BF16)
    ri = jax.lax.broadcasted_iota(jnp.int32, (CH, CH), 0)
    ci = jax.lax.broadcasted_iota(jnp.int32, (CH, CH), 1)
    blk = (ri // N == ci // N).astype(_BF16)
    for c in range(P // CH):
        rows = _dot(g_ref[c * CH:(c + 1) * CH, :].astype(_BF16), T)
        gbd_ref[c * CH:(c + 1) * CH, :] = rows.astype(_BF16) * blk

    rl = jax.lax.broadcasted_iota(jnp.int32, (N, EW), 1)
    rw = jax.lax.broadcasted_iota(jnp.int32, (N, EW), 0)
    R = (rl // D_E == rw).astype(_BF16)
    sl = jax.lax.broadcasted_iota(jnp.int32, (EW, D_E), 0)
    sj = jax.lax.broadcasted_iota(jnp.int32, (EW, D_E), 1)
    S = (sl % D_E == sj).astype(_BF16)
    g_rep = _dot(g.astype(_BF16), R)
    prod = g_rep.astype(_BF16) * e_ref[...].astype(_BF16)
    m_e = _dot(prod, S)
    m_e_bf = m_e.astype(_BF16)

    dmask = [(deg == float(d)).astype(_BF16) for d in range(NDEG)]

    el = jax.lax.broadcasted_iota(jnp.int32, (D_E, 8 * D_E), 1)
    ei = jax.lax.broadcasted_iota(jnp.int32, (D_E, 8 * D_E), 0)
    Erep = (el % D_E == ei).astype(_BF16)
    qpat = (jax.lax.broadcasted_iota(jnp.int32, (1, 8 * D_E), 1)
            // D_E).astype(_F32)

    m_e8 = _dot(m_e_bf, Erep).astype(_BF16)
    epiece = [m_e8 * (deg - float(8 * q) == qpat).astype(_BF16)
              for q in range(4)]
    e32 = m_e_bf * (deg == float(32)).astype(_BF16)

    def readout(h_l, W_ref):
        mask = (jnp.sum(h_l, axis=1, keepdims=True) != 0).astype(_F32)
        hm = h_l * mask
        hsum = jnp.sum(hm.reshape(B, N, OUT), axis=1)
        return _dot(hsum, W_ref[...])

    h = h_ref[...]
    aux = readout(h, W0_ref)

    for H_ref, W_ref in ((H0_ref, W1_ref), (H1_ref, W2_ref), (H2_ref, W3_ref)):
        h_bf = h.astype(_BF16)
        mh_cs = [_dot(gbd_ref[c * CH:(c + 1) * CH, :],
                      h_bf[c * CH:(c + 1) * CH, :]) for c in range(P // CH)]
        m_h = jnp.concatenate(mh_cs, axis=0)
        m_h_bf = m_h.astype(_BF16)

        def hpieces(lo, hi):
            return [m_h_bf * dmask[d] for d in range(lo, hi)]

        def hbank(ref, lo, hi):
            return jnp.concatenate(
                [ref[d, :D_IN, :] for d in range(lo, hi)],
                axis=0).astype(_BF16)

        def ebank(ref, lo, hi):
            return jnp.concatenate(
                [ref[d, D_IN:, :] for d in range(lo, hi)],
                axis=0).astype(_BF16)

        lhs_c = jnp.concatenate(
            hpieces(C_LO, C_HI) + [epiece[1], epiece[2]], axis=1)
        rhs_c = jnp.concatenate(
            [hbank(H_ref, C_LO, C_HI), ebank(H_ref, C_LO, C_HI)], axis=0)
        acc_ref[...] = _dot(lhs_c, rhs_c)

        @pl.when(deg_min < 4.0)
        def _():
            lhs = jnp.concatenate(hpieces(0, 4), axis=1)
            acc_ref[...] += _dot(lhs, hbank(H_ref, 0, 4))

        @pl.when(any47 > 0.0)
        def _():
            lhs = jnp.concatenate(hpieces(4, C_LO), axis=1)
            acc_ref[...] += _dot(lhs, hbank(H_ref, 4, C_LO))

        @pl.when(deg_min < float(C_LO))
        def _():
            acc_ref[...] += _dot(epiece[0], ebank(H_ref, 0, C_LO))

        @pl.when(any2427 > 0.0)
        def _():
            lhs = jnp.concatenate(hpieces(C_HI, 28), axis=1)
            acc_ref[...] += _dot(lhs, hbank(H_ref, C_HI, 28))

        @pl.when(any2831 > 0.0)
        def _():
            lhs = jnp.concatenate(hpieces(28, 32), axis=1)
            acc_ref[...] += _dot(lhs, hbank(H_ref, 28, 32))

        @pl.when(deg_max >= float(C_HI))
        def _():
            acc_ref[...] += _dot(epiece[3], ebank(H_ref, C_HI, 32))

        @pl.when(deg_max == float(32))
        def _():
            lhs = jnp.concatenate(
                [m_h_bf * dmask[32], e32], axis=1)
            acc_ref[...] += _dot(lhs, H_ref[32].astype(_BF16))

        h = jax.nn.sigmoid(acc_ref[...])
        aux = aux + readout(h, W_ref)

    s = jax.nn.softmax(aux, axis=1)
    x = jax.nn.relu(_dot(s, nW0_ref[...]) + nb0_ref[...])
    x = jax.nn.relu(_dot(x, nW1_ref[...]) + nb1_ref[...])
    x = jax.nn.relu(_dot(x, nW2_ref[...]) + nb2_ref[...])
    x = jax.nn.sigmoid(_dot(x, nW3_ref[...]) + nb3_ref[...])
    out_ref[...] = jax.nn.softmax(x, axis=1)


@functools.partial(jax.jit, static_argnames=("interpret",))
def _run(g, h_in, e, H0, H1, H2, W0, W1, W2, W3,
         nW0, nb0, nW1, nb1, nW2, nb2, nW3, nb3, interpret=False):
    g2 = g.reshape(P, N)
    e2 = e.reshape(P, EW)
    h2 = h_in.reshape(P, D_IN)
    return pl.pallas_call(
        _gnn_kernel,
        out_shape=jax.ShapeDtypeStruct((B, TGT), _F32),
        scratch_shapes=[pltpu.VMEM((P, CH), _BF16),
                        pltpu.VMEM((P, OUT), _F32)],
        interpret=interpret,
    )(g2, e2, h2, H0, H1, H2, W0, W1, W2, W3,
      nW0, nb0.reshape(1, -1), nW1, nb1.reshape(1, -1),
      nW2, nb2.reshape(1, -1), nW3, nb3.reshape(1, -1))


def kernel(g, h_in, e, H0, H1, H2, W0, W1, W2, W3,
           nW0, nb0, nW1, nb1, nW2, nb2, nW3, nb3):
    return _run(g, h_in, e, H0, H1, H2, W0, W1, W2, W3,
                nW0, nb0, nW1, nb1, nW2, nb2, nW3, nb3)
